# baseline (speedup 1.0000x reference)
import functools

import jax
import jax.numpy as jnp
from jax import lax
from jax.experimental import pallas as pl
from jax.experimental.pallas import tpu as pltpu
from jax.experimental.pallas import tpu_sc as plsc

NUM_FEATURES = 2
B = 2
S = 128
D = 64
REPS = 4
H = NUM_FEATURES * REPS
NW = 32
ROWS_PER_W = S // NW
BLK = 16


def _sc_gather_body(idx_hbm, kv0, kv1, inter, idx_v, b0, b1,
                    gs0, gs1, ss0, ss1):
    wid = lax.axis_index("s") * 2 + lax.axis_index("c")
    bufs = (b0, b1)
    gsems = (gs0, gs1)
    ssems = (ss0, ss1)
    tables = (kv0, kv1)
    units = [(f, b, half) for f in range(NUM_FEATURES) for b in range(B)
             for half in range(2)]
    pltpu.sync_copy(idx_hbm.at[wid], idx_v)
    gathers = [None] * 2
    scatters = [None] * 2

    def issue_gathers(t):
        f, b, half = units[t]
        slot = t % 2
        gathers[slot] = [
            pltpu.async_copy(
                tables[f].at[idx_v.at[(f * B + b) * ROWS_PER_W + 2 * half + c]],
                bufs[slot].at[c], gsems[slot])
            for c in range(2)
        ]

    issue_gathers(0)
    for t in range(len(units)):
        f, b, half = units[t]
        slot = t % 2
        for g in gathers[slot]:
            g.wait()
        if t + 1 < len(units):
            nslot = (t + 1) % 2
            if scatters[nslot] is not None:
                scatters[nslot].wait()
                scatters[nslot] = None
            issue_gathers(t + 1)
        i0 = wid * ROWS_PER_W + 2 * half
        scatters[slot] = pltpu.async_copy(
            bufs[slot], inter.at[f, b, pl.ds(i0, 2), :, :],
            ssems[slot])
    for s in scatters:
        if s is not None:
            s.wait()


def _tc_replicate_body(inter_ref, k_ref, v_ref):
    for f in range(NUM_FEATURES):
        x = inter_ref[f, 0]
        k = x[:, :, :D]
        v = x[:, :, D:]
        for r in range(REPS):
            h = REPS * f + r
            k_ref[0, h] = k
            v_ref[0, h] = v


@jax.jit
def _tree_rel_pos(idx_perm, kv0, kv1):
    mesh = plsc.VectorSubcoreMesh(core_axis_name="c", subcore_axis_name="s")
    inter_sds = jax.ShapeDtypeStruct((NUM_FEATURES, B, S, S, 2 * D),
                                     jnp.float32)
    buf = pltpu.VMEM((2, S, 2 * D), jnp.float32)
    sc_run = functools.partial(
        pl.kernel,
        out_type=inter_sds,
        mesh=mesh,
        scratch_types=[pltpu.VMEM((NUM_FEATURES * B * ROWS_PER_W, S),
                                  jnp.int32)]
        + [buf] * 2 + [pltpu.SemaphoreType.DMA] * 4,
    )(_sc_gather_body)
    inter = sc_run(idx_perm, kv0, kv1)

    out_sds = jax.ShapeDtypeStruct((B, H, S, S, D), jnp.float32)
    k_out, v_out = pl.pallas_call(
        _tc_replicate_body,
        grid=(B, S // BLK),
        in_specs=[pl.BlockSpec((NUM_FEATURES, 1, BLK, S, 2 * D),
                               lambda b, i: (0, b, i, 0, 0))],
        out_specs=[
            pl.BlockSpec((1, H, BLK, S, D), lambda b, i: (b, 0, i, 0, 0)),
            pl.BlockSpec((1, H, BLK, S, D), lambda b, i: (b, 0, i, 0, 0)),
        ],
        out_shape=[out_sds, out_sds],
    )(inter)
    return k_out, v_out


def kernel(inputs, emb0, emb1):
    scale = float(D) ** 0.5
    idx_perm = jnp.transpose(
        inputs.reshape(NUM_FEATURES, B, NW, ROWS_PER_W, S),
        (2, 0, 1, 3, 4)).reshape(NW, NUM_FEATURES * B * ROWS_PER_W, S)
    k_out, v_out = _tree_rel_pos(idx_perm, emb0 * scale, emb1 * scale)
    return (k_out, v_out)


# speedup vs baseline: 1.2536x; 1.2536x over previous
import functools

import jax
import jax.numpy as jnp
from jax import lax
from jax.experimental import pallas as pl
from jax.experimental.pallas import tpu as pltpu
from jax.experimental.pallas import tpu_sc as plsc

NUM_FEATURES = 2
B = 2
S = 128
D = 64
REPS = 4
H = NUM_FEATURES * REPS
NW = 32
ROWS_PER_W = S // NW
BLK = 16


def _sc_gather_body(idx_hbm, kv0, kv1, inter, idx_v, b0, b1,
                    gs0, gs1, ss0, ss1):
    wid = lax.axis_index("s") * 2 + lax.axis_index("c")
    bufs = (b0, b1)
    gsems = (gs0, gs1)
    ssems = (ss0, ss1)
    tables = (kv0, kv1)
    units = [(f, b, half) for f in range(NUM_FEATURES) for b in range(B)
             for half in range(2)]
    pltpu.sync_copy(idx_hbm.at[wid], idx_v)
    gathers = [None] * 2
    scatters = [None] * 2

    def issue_gathers(t):
        f, b, half = units[t]
        slot = t % 2
        gathers[slot] = [
            pltpu.async_copy(
                tables[f].at[idx_v.at[(f * B + b) * ROWS_PER_W + 2 * half + c]],
                bufs[slot].at[c], gsems[slot])
            for c in range(2)
        ]

    issue_gathers(0)
    for t in range(len(units)):
        f, b, half = units[t]
        slot = t % 2
        for g in gathers[slot]:
            g.wait()
        if t + 1 < len(units):
            nslot = (t + 1) % 2
            if scatters[nslot] is not None:
                scatters[nslot].wait()
                scatters[nslot] = None
            issue_gathers(t + 1)
        i0 = wid * ROWS_PER_W + 2 * half
        scatters[slot] = pltpu.async_copy(
            bufs[slot], inter.at[f, b, pl.ds(i0, 2), :, :],
            ssems[slot])
    for s in scatters:
        if s is not None:
            s.wait()


def _tc_replicate_body(inter_ref, k_ref, v_ref):
    for f in range(NUM_FEATURES):
        x = inter_ref[f, 0]
        y = x.reshape(BLK, S, 2 * D)
        k = y[:, :, :D]
        v = y[:, :, D:]
        for r in range(REPS):
            h = REPS * f + r
            k_ref[0, h] = k
            v_ref[0, h] = v


@jax.jit
def _tree_rel_pos(idx_perm, kv0, kv1):
    mesh = plsc.VectorSubcoreMesh(core_axis_name="c", subcore_axis_name="s")
    inter_sds = jax.ShapeDtypeStruct((NUM_FEATURES, B, S, S // 2, 4 * D),
                                     jnp.float32)
    buf = pltpu.VMEM((2, S // 2, 4 * D), jnp.float32)
    sc_run = functools.partial(
        pl.kernel,
        out_type=inter_sds,
        mesh=mesh,
        scratch_types=[pltpu.VMEM((NUM_FEATURES * B * ROWS_PER_W, S // 2),
                                  jnp.int32)]
        + [buf] * 2 + [pltpu.SemaphoreType.DMA] * 4,
    )(_sc_gather_body)
    inter = sc_run(idx_perm, kv0, kv1)

    out_sds = jax.ShapeDtypeStruct((B, H, S, S, D), jnp.float32)
    k_out, v_out = pl.pallas_call(
        _tc_replicate_body,
        grid=(B, S // BLK),
        in_specs=[pl.BlockSpec((NUM_FEATURES, 1, BLK, S // 2, 4 * D),
                               lambda b, i: (0, b, i, 0, 0))],
        out_specs=[
            pl.BlockSpec((1, H, BLK, S, D), lambda b, i: (b, 0, i, 0, 0)),
            pl.BlockSpec((1, H, BLK, S, D), lambda b, i: (b, 0, i, 0, 0)),
        ],
        out_shape=[out_sds, out_sds],
    )(inter)
    return k_out, v_out


def kernel(inputs, emb0, emb1):
    scale = float(D) ** 0.5
    vocab = emb0.shape[0]
    pidx = inputs[..., 0::2] * vocab + inputs[..., 1::2]
    idx_perm = jnp.transpose(
        pidx.reshape(NUM_FEATURES, B, NW, ROWS_PER_W, S // 2),
        (2, 0, 1, 3, 4)).reshape(NW, NUM_FEATURES * B * ROWS_PER_W, S // 2)

    def pair_table(t):
        t1 = jnp.broadcast_to(t[:, None, :], (vocab, vocab, 2 * D))
        t2 = jnp.broadcast_to(t[None, :, :], (vocab, vocab, 2 * D))
        return jnp.concatenate([t1, t2], -1).reshape(vocab * vocab, 4 * D)

    k_out, v_out = _tree_rel_pos(idx_perm, pair_table(emb0 * scale),
                                 pair_table(emb1 * scale))
    return (k_out, v_out)
